# baseline (speedup 1.0000x reference)
import dataclasses
import functools

import jax
import jax.numpy as jnp
from jax import lax
from jax.experimental import pallas as pl
from jax.experimental.pallas import tpu as pltpu
from jax.experimental.pallas import tpu_sc as plsc

N = 10000
D = 128
E = 320000
NC = 2
NS = 16
NW = NC * NS
CHUNK = 80
NCHUNK = E // CHUNK
CPW = NCHUNK // NW
ROWS_PAD = 10240
RPS = ROWS_PAD // NS
DUMP = N
AGG_ROWS = ROWS_PAD
CH_S = 112
CPW_S = -(-(E // NW) // CH_S)
EPAD = NW * CPW_S * CH_S


def _sc_compiler_params():
    cp = pltpu.CompilerParams()
    if "needs_layout_passes" in pltpu.CompilerParams.__dataclass_fields__:
        cp = dataclasses.replace(cp, needs_layout_passes=False)
    return cp


@functools.cache
def _get_sc_prep():
    mesh = plsc.VectorSubcoreMesh(core_axis_name="c", subcore_axis_name="s")
    return functools.partial(
        pl.kernel,
        out_type=[
            jax.ShapeDtypeStruct((NW, CPW, CHUNK), jnp.int32),
            jax.ShapeDtypeStruct((NW, ROWS_PAD), jnp.float32),
            jax.ShapeDtypeStruct((NW, ROWS_PAD), jnp.float32),
        ],
        mesh=mesh,
        compiler_params=_sc_compiler_params(),
        scratch_types=[
            pltpu.VMEM((CPW, CHUNK), jnp.int32),
            pltpu.VMEM((CPW, CHUNK), jnp.int32),
            pltpu.VMEM((CPW, CHUNK), jnp.int32),
            pltpu.VMEM((ROWS_PAD,), jnp.float32),
            pltpu.VMEM((ROWS_PAD,), jnp.float32),
        ],
    )(_sc_prep_body)


def _sc_prep_body(src_hbm, dst_hbm, dstm_hbm, dego_hbm, degi_hbm,
                  src_v, dst_v, dstm_v, dego_p, degi_p):
    c = lax.axis_index("c")
    s = lax.axis_index("s")
    wid = c * NS + s

    @pl.loop(0, ROWS_PAD // 16)
    def _(i):
        z = jnp.zeros((16,), jnp.float32)
        dego_p[pl.ds(i * 16, 16)] = z
        degi_p[pl.ds(i * 16, 16)] = z

    pltpu.sync_copy(src_hbm.at[wid], src_v)
    pltpu.sync_copy(dst_hbm.at[wid], dst_v)
    ones16 = jnp.full((16,), 1.0, jnp.float32)
    dump = jnp.full((16,), DUMP, jnp.int32)

    @pl.loop(0, CPW)
    def _(i):
        for j in range(CHUNK // 16):
            sl = pl.ds(j * 16, 16)
            sv = src_v[i, sl]
            dv = dst_v[i, sl]
            m = sv == dv
            sm = jnp.where(m, dump, sv)
            dm = jnp.where(m, dump, dv)
            dstm_v[i, sl] = dm
            plsc.addupdate_scatter(dego_p, [sm], ones16)
            plsc.addupdate_scatter(degi_p, [dm], ones16)

    pltpu.sync_copy(dstm_v, dstm_hbm.at[wid])
    pltpu.sync_copy(dego_p, dego_hbm.at[wid])
    pltpu.sync_copy(degi_p, degi_hbm.at[wid])


@functools.cache
def _get_sc_scatter():
    mesh = plsc.VectorSubcoreMesh(core_axis_name="c", subcore_axis_name="s")
    return functools.partial(
        pl.kernel,
        out_type=jax.ShapeDtypeStruct((NC, ROWS_PAD, D), jnp.float32),
        mesh=mesh,
        scratch_types=[
            pltpu.VMEM((CPW_S, CH_S), jnp.int32),
            pltpu.VMEM((CPW_S, CH_S), jnp.int32),
            pltpu.VMEM((CH_S, D), jnp.float32),
            pltpu.VMEM((CH_S, D), jnp.float32),
            pltpu.SemaphoreType.DMA,
            pltpu.SemaphoreType.DMA,
            pltpu.VMEM_SHARED((AGG_ROWS, D), jnp.float32),
        ],
    )(_sc_scatter_body)


def _sc_scatter_body(f_hbm, src_hbm, dstm_hbm, part_hbm,
                     src_v, dstm_v, buf0, buf1, sem0, sem1, agg_sh):
    c = lax.axis_index("c")
    s = lax.axis_index("s")
    wid = c * NS + s
    base = s * RPS

    @pl.loop(0, 64)
    def _(i):
        for j in range(D // 16):
            buf0[i, pl.ds(j * 16, 16)] = jnp.zeros((16,), jnp.float32)

    @pl.loop(0, RPS // 64)
    def _(j):
        pltpu.sync_copy(buf0.at[pl.ds(0, 64)],
                        agg_sh.at[pl.ds(base + j * 64, 64)])

    plsc.subcore_barrier()

    pltpu.sync_copy(src_hbm.at[wid], src_v)
    pltpu.sync_copy(dstm_hbm.at[wid], dstm_v)

    @pl.loop(0, CPW_S)
    def _(i):
        pltpu.sync_copy(f_hbm.at[src_v.at[i]], buf0)
        pltpu.sync_copy(buf0, agg_sh.at[dstm_v.at[i]], add=True)

    plsc.subcore_barrier()

    sl = pl.ds(base, RPS)
    pltpu.sync_copy(agg_sh.at[sl], part_hbm.at[c, sl])


_BR = 1024
_NBLK = ROWS_PAD // _BR


def _tc_setup_body(x_ref, dego_ref, degi_ref, W_ref, f_ref, ns_ref, nd_ref):
    dego = jnp.sum(dego_ref[...], axis=0)[:, None] + 1.0
    degi = jnp.sum(degi_ref[...], axis=0)[:, None] + 1.0
    ns = lax.rsqrt(dego)
    nd = lax.rsqrt(degi)
    ns_ref[...] = ns
    nd_ref[...] = nd
    f_ref[...] = jnp.dot(x_ref[...] * ns, W_ref[...],
                         preferred_element_type=jnp.float32)


def _tc_setup(x, dego, degi, W0):
    return pl.pallas_call(
        _tc_setup_body,
        grid=(_NBLK,),
        in_specs=[
            pl.BlockSpec((_BR, D), lambda i: (i, 0)),
            pl.BlockSpec((NW, _BR), lambda i: (0, i)),
            pl.BlockSpec((NW, _BR), lambda i: (0, i)),
            pl.BlockSpec((D, D), lambda i: (0, 0)),
        ],
        out_specs=[
            pl.BlockSpec((_BR, D), lambda i: (i, 0)),
            pl.BlockSpec((_BR, 1), lambda i: (i, 0)),
            pl.BlockSpec((_BR, 1), lambda i: (i, 0)),
        ],
        out_shape=[
            jax.ShapeDtypeStruct((ROWS_PAD, D), jnp.float32),
            jax.ShapeDtypeStruct((ROWS_PAD, 1), jnp.float32),
            jax.ShapeDtypeStruct((ROWS_PAD, 1), jnp.float32),
        ],
    )(x, dego, degi, W0)


def _gate(h, nxt, Wg1, Wg2, bg2):
    z = (jnp.dot(h, Wg1, preferred_element_type=jnp.float32)
         + jnp.dot(nxt, Wg2, preferred_element_type=jnp.float32) + bg2)
    scale = jax.nn.sigmoid(z)
    return h * scale + nxt * (1.0 - scale)


def _tc_post_body(p_ref, f_ref, h_ref, nd_ref, ns_ref, b_ref, Wg1_ref,
                  Wg2_ref, bg2_ref, Wn_ref, hn_ref, fn_ref):
    agg = (p_ref[0] + p_ref[1] + f_ref[...]) * nd_ref[...] + b_ref[...]
    nxt = jnp.where(agg > 0, agg, 0.01 * agg)
    hn = _gate(h_ref[...], nxt, Wg1_ref[...], Wg2_ref[...], bg2_ref[...])
    hn_ref[...] = hn
    fn_ref[...] = jnp.dot(hn * ns_ref[...], Wn_ref[...],
                          preferred_element_type=jnp.float32)


def _tc_post_final_body(p_ref, f_ref, h_ref, nd_ref, b_ref, Wg1_ref,
                        Wg2_ref, bg2_ref, hn_ref):
    agg = (p_ref[0] + p_ref[1] + f_ref[...]) * nd_ref[...] + b_ref[...]
    nxt = jnp.where(agg > 0, agg, 0.01 * agg)
    hn_ref[...] = _gate(h_ref[...], nxt, Wg1_ref[...], Wg2_ref[...],
                        bg2_ref[...])


_row_spec = pl.BlockSpec((_BR, D), lambda i: (i, 0))
_part_spec = pl.BlockSpec((NC, _BR, D), lambda i: (0, i, 0))
_norm_spec = pl.BlockSpec((_BR, 1), lambda i: (i, 0))
_w_spec = pl.BlockSpec((D, D), lambda i: (0, 0))
_b_spec = pl.BlockSpec((1, D), lambda i: (0, 0))


def _tc_post(part, f, h, nd, ns, b, Wg1, Wg2, bg2, Wn):
    return pl.pallas_call(
        _tc_post_body,
        grid=(_NBLK,),
        in_specs=[_part_spec, _row_spec, _row_spec, _norm_spec, _norm_spec,
                  _b_spec, _w_spec, _w_spec, _b_spec, _w_spec],
        out_specs=[_row_spec, _row_spec],
        out_shape=[
            jax.ShapeDtypeStruct((ROWS_PAD, D), jnp.float32),
            jax.ShapeDtypeStruct((ROWS_PAD, D), jnp.float32),
        ],
    )(part, f, h, nd, ns, b, Wg1, Wg2, bg2, Wn)


def _tc_post_final(part, f, h, nd, b, Wg1, Wg2, bg2):
    return pl.pallas_call(
        _tc_post_final_body,
        grid=(_NBLK,),
        in_specs=[_part_spec, _row_spec, _row_spec, _norm_spec,
                  _b_spec, _w_spec, _w_spec, _b_spec],
        out_specs=_row_spec,
        out_shape=jax.ShapeDtypeStruct((ROWS_PAD, D), jnp.float32),
    )(part, f, h, nd, b, Wg1, Wg2, bg2)


def kernel(x, edge_index, W0, b0, W1, b1, W2, b2, Wg1, Wg2, bg2):
    src = edge_index[0].reshape(NW, CPW, CHUNK)
    dst = edge_index[1].reshape(NW, CPW, CHUNK)
    dstm, dego, degi = _get_sc_prep()(src, dst)
    xp = jnp.pad(x, ((0, ROWS_PAD - N), (0, 0)))
    f, ns, nd = _tc_setup(xp, dego, degi, W0)
    h = xp
    bs = [b0.reshape(1, D), b1.reshape(1, D), b2.reshape(1, D)]
    bg2r = bg2.reshape(1, D)
    next_W = [W1, W2, None]
    if EPAD == E:
        src_s = edge_index[0].reshape(NW, CPW_S, CH_S)
        dstm_s = dstm.reshape(NW, CPW_S, CH_S)
    else:
        padi = jnp.zeros((EPAD - E,), jnp.int32)
        src_s = jnp.concatenate([edge_index[0], padi]).reshape(
            NW, CPW_S, CH_S)
        dstm_s = jnp.concatenate(
            [dstm.reshape(E), jnp.full((EPAD - E,), DUMP, jnp.int32)]
        ).reshape(NW, CPW_S, CH_S)
    for l in range(3):
        part = _get_sc_scatter()(f, src_s, dstm_s)
        if l < 2:
            h, f = _tc_post(part, f, h, nd, ns, bs[l], Wg1, Wg2, bg2r,
                            next_W[l])
        else:
            h = _tc_post_final(part, f, h, nd, bs[l], Wg1, Wg2, bg2r)
    return h[:N]


# speedup vs baseline: 1.5227x; 1.5227x over previous
import dataclasses
import functools

import jax
import jax.numpy as jnp
from jax import lax
from jax.experimental import pallas as pl
from jax.experimental.pallas import tpu as pltpu
from jax.experimental.pallas import tpu_sc as plsc

N = 10000
D = 128
E = 320000
NC = 2
NS = 16
NW = NC * NS
CHUNK = 80
NCHUNK = E // CHUNK
CPW = NCHUNK // NW
ROWS_PAD = 10240
RPS = ROWS_PAD // NS
DUMP = N
AGG_ROWS = ROWS_PAD
CH_S = 125
CPW_S = -(-(E // NW) // CH_S)
EPAD = NW * CPW_S * CH_S


def _sc_compiler_params():
    cp = pltpu.CompilerParams()
    if "needs_layout_passes" in pltpu.CompilerParams.__dataclass_fields__:
        cp = dataclasses.replace(cp, needs_layout_passes=False)
    return cp


@functools.cache
def _get_sc_prep():
    mesh = plsc.VectorSubcoreMesh(core_axis_name="c", subcore_axis_name="s")
    return functools.partial(
        pl.kernel,
        out_type=[
            jax.ShapeDtypeStruct((NW, CPW, CHUNK), jnp.int32),
            jax.ShapeDtypeStruct((NW, ROWS_PAD), jnp.float32),
            jax.ShapeDtypeStruct((NW, ROWS_PAD), jnp.float32),
        ],
        mesh=mesh,
        compiler_params=_sc_compiler_params(),
        scratch_types=[
            pltpu.VMEM((CPW, CHUNK), jnp.int32),
            pltpu.VMEM((CPW, CHUNK), jnp.int32),
            pltpu.VMEM((CPW, CHUNK), jnp.int32),
            pltpu.VMEM((ROWS_PAD,), jnp.float32),
            pltpu.VMEM((ROWS_PAD,), jnp.float32),
        ],
    )(_sc_prep_body)


def _sc_prep_body(src_hbm, dst_hbm, dstm_hbm, dego_hbm, degi_hbm,
                  src_v, dst_v, dstm_v, dego_p, degi_p):
    c = lax.axis_index("c")
    s = lax.axis_index("s")
    wid = c * NS + s

    @pl.loop(0, ROWS_PAD // 16)
    def _(i):
        z = jnp.zeros((16,), jnp.float32)
        dego_p[pl.ds(i * 16, 16)] = z
        degi_p[pl.ds(i * 16, 16)] = z

    pltpu.sync_copy(src_hbm.at[wid], src_v)
    pltpu.sync_copy(dst_hbm.at[wid], dst_v)
    ones16 = jnp.full((16,), 1.0, jnp.float32)
    dump = jnp.full((16,), DUMP, jnp.int32)

    @pl.loop(0, CPW)
    def _(i):
        for j in range(CHUNK // 16):
            sl = pl.ds(j * 16, 16)
            sv = src_v[i, sl]
            dv = dst_v[i, sl]
            m = sv == dv
            sm = jnp.where(m, dump, sv)
            dm = jnp.where(m, dump, dv)
            dstm_v[i, sl] = dm
            plsc.addupdate_scatter(dego_p, [sm], ones16)
            plsc.addupdate_scatter(degi_p, [dm], ones16)

    pltpu.sync_copy(dstm_v, dstm_hbm.at[wid])
    pltpu.sync_copy(dego_p, dego_hbm.at[wid])
    pltpu.sync_copy(degi_p, degi_hbm.at[wid])


@functools.cache
def _get_sc_scatter():
    mesh = plsc.VectorSubcoreMesh(core_axis_name="c", subcore_axis_name="s")
    return functools.partial(
        pl.kernel,
        out_type=jax.ShapeDtypeStruct((NC, ROWS_PAD, D), jnp.float32),
        mesh=mesh,
        scratch_types=[
            pltpu.VMEM((CPW_S, CH_S), jnp.int32),
            pltpu.VMEM((CPW_S, CH_S), jnp.int32),
            pltpu.VMEM((CH_S, D), jnp.float32),
            pltpu.VMEM((CH_S, D), jnp.float32),
            pltpu.SemaphoreType.DMA,
            pltpu.SemaphoreType.DMA,
            pltpu.VMEM_SHARED((AGG_ROWS, D), jnp.float32),
        ],
    )(_sc_scatter_body)


def _sc_scatter_body(f_hbm, src_hbm, dstm_hbm, part_hbm,
                     src_v, dstm_v, buf0, buf1, sem0, sem1, agg_sh):
    c = lax.axis_index("c")
    s = lax.axis_index("s")
    wid = c * NS + s
    base = s * RPS

    @pl.loop(0, 64)
    def _(i):
        for j in range(D // 16):
            buf0[i, pl.ds(j * 16, 16)] = jnp.zeros((16,), jnp.float32)

    @pl.loop(0, RPS // 64)
    def _(j):
        pltpu.sync_copy(buf0.at[pl.ds(0, 64)],
                        agg_sh.at[pl.ds(base + j * 64, 64)])

    plsc.subcore_barrier()

    pltpu.sync_copy(src_hbm.at[wid], src_v)
    pltpu.sync_copy(dstm_hbm.at[wid], dstm_v)

    @pl.loop(0, CPW_S)
    def _(i):
        pltpu.sync_copy(f_hbm.at[src_v.at[i]], buf0)
        pltpu.sync_copy(buf0, agg_sh.at[dstm_v.at[i]], add=True)

    plsc.subcore_barrier()

    sl = pl.ds(base, RPS)
    pltpu.sync_copy(agg_sh.at[sl], part_hbm.at[c, sl])


_BR = 1024
_NBLK = ROWS_PAD // _BR


def _tc_setup_body(x_ref, dego_ref, degi_ref, W_ref, f_ref, ns_ref, nd_ref):
    dego = jnp.sum(dego_ref[...], axis=0)[:, None] + 1.0
    degi = jnp.sum(degi_ref[...], axis=0)[:, None] + 1.0
    ns = lax.rsqrt(dego)
    nd = lax.rsqrt(degi)
    ns_ref[...] = ns
    nd_ref[...] = nd
    f_ref[...] = jnp.dot(x_ref[...] * ns, W_ref[...],
                         preferred_element_type=jnp.float32)


def _tc_setup(x, dego, degi, W0):
    return pl.pallas_call(
        _tc_setup_body,
        grid=(_NBLK,),
        in_specs=[
            pl.BlockSpec((_BR, D), lambda i: (i, 0)),
            pl.BlockSpec((NW, _BR), lambda i: (0, i)),
            pl.BlockSpec((NW, _BR), lambda i: (0, i)),
            pl.BlockSpec((D, D), lambda i: (0, 0)),
        ],
        out_specs=[
            pl.BlockSpec((_BR, D), lambda i: (i, 0)),
            pl.BlockSpec((_BR, 1), lambda i: (i, 0)),
            pl.BlockSpec((_BR, 1), lambda i: (i, 0)),
        ],
        out_shape=[
            jax.ShapeDtypeStruct((ROWS_PAD, D), jnp.float32),
            jax.ShapeDtypeStruct((ROWS_PAD, 1), jnp.float32),
            jax.ShapeDtypeStruct((ROWS_PAD, 1), jnp.float32),
        ],
    )(x, dego, degi, W0)


def _gate(h, nxt, Wg1, Wg2, bg2):
    z = (jnp.dot(h, Wg1, preferred_element_type=jnp.float32)
         + jnp.dot(nxt, Wg2, preferred_element_type=jnp.float32) + bg2)
    scale = jax.nn.sigmoid(z)
    return h * scale + nxt * (1.0 - scale)


def _tc_post_body(p_ref, f_ref, h_ref, nd_ref, ns_ref, b_ref, Wg1_ref,
                  Wg2_ref, bg2_ref, Wn_ref, hn_ref, fn_ref):
    agg = (p_ref[0] + p_ref[1] + f_ref[...]) * nd_ref[...] + b_ref[...]
    nxt = jnp.where(agg > 0, agg, 0.01 * agg)
    hn = _gate(h_ref[...], nxt, Wg1_ref[...], Wg2_ref[...], bg2_ref[...])
    hn_ref[...] = hn
    fn_ref[...] = jnp.dot(hn * ns_ref[...], Wn_ref[...],
                          preferred_element_type=jnp.float32)


def _tc_post_final_body(p_ref, f_ref, h_ref, nd_ref, b_ref, Wg1_ref,
                        Wg2_ref, bg2_ref, hn_ref):
    agg = (p_ref[0] + p_ref[1] + f_ref[...]) * nd_ref[...] + b_ref[...]
    nxt = jnp.where(agg > 0, agg, 0.01 * agg)
    hn_ref[...] = _gate(h_ref[...], nxt, Wg1_ref[...], Wg2_ref[...],
                        bg2_ref[...])


_row_spec = pl.BlockSpec((_BR, D), lambda i: (i, 0))
_part_spec = pl.BlockSpec((NC, _BR, D), lambda i: (0, i, 0))
_norm_spec = pl.BlockSpec((_BR, 1), lambda i: (i, 0))
_w_spec = pl.BlockSpec((D, D), lambda i: (0, 0))
_b_spec = pl.BlockSpec((1, D), lambda i: (0, 0))


def _tc_post(part, f, h, nd, ns, b, Wg1, Wg2, bg2, Wn):
    return pl.pallas_call(
        _tc_post_body,
        grid=(_NBLK,),
        in_specs=[_part_spec, _row_spec, _row_spec, _norm_spec, _norm_spec,
                  _b_spec, _w_spec, _w_spec, _b_spec, _w_spec],
        out_specs=[_row_spec, _row_spec],
        out_shape=[
            jax.ShapeDtypeStruct((ROWS_PAD, D), jnp.float32),
            jax.ShapeDtypeStruct((ROWS_PAD, D), jnp.float32),
        ],
    )(part, f, h, nd, ns, b, Wg1, Wg2, bg2, Wn)


def _tc_post_final(part, f, h, nd, b, Wg1, Wg2, bg2):
    return pl.pallas_call(
        _tc_post_final_body,
        grid=(_NBLK,),
        in_specs=[_part_spec, _row_spec, _row_spec, _norm_spec,
                  _b_spec, _w_spec, _w_spec, _b_spec],
        out_specs=_row_spec,
        out_shape=jax.ShapeDtypeStruct((ROWS_PAD, D), jnp.float32),
    )(part, f, h, nd, b, Wg1, Wg2, bg2)


def kernel(x, edge_index, W0, b0, W1, b1, W2, b2, Wg1, Wg2, bg2):
    src = edge_index[0].reshape(NW, CPW, CHUNK)
    dst = edge_index[1].reshape(NW, CPW, CHUNK)
    dstm, dego, degi = _get_sc_prep()(src, dst)
    xp = jnp.pad(x, ((0, ROWS_PAD - N), (0, 0)))
    f, ns, nd = _tc_setup(xp, dego, degi, W0)
    h = xp
    bs = [b0.reshape(1, D), b1.reshape(1, D), b2.reshape(1, D)]
    bg2r = bg2.reshape(1, D)
    next_W = [W1, W2, None]
    if EPAD == E:
        src_s = edge_index[0].reshape(NW, CPW_S, CH_S)
        dstm_s = dstm.reshape(NW, CPW_S, CH_S)
    else:
        padi = jnp.zeros((EPAD - E,), jnp.int32)
        src_s = jnp.concatenate([edge_index[0], padi]).reshape(
            NW, CPW_S, CH_S)
        dstm_s = jnp.concatenate(
            [dstm.reshape(E), jnp.full((EPAD - E,), DUMP, jnp.int32)]
        ).reshape(NW, CPW_S, CH_S)
    for l in range(3):
        part = _get_sc_scatter()(f, src_s, dstm_s)
        if l < 2:
            h, f = _tc_post(part, f, h, nd, ns, bs[l], Wg1, Wg2, bg2r,
                            next_W[l])
        else:
            h = _tc_post_final(part, f, h, nd, bs[l], Wg1, Wg2, bg2r)
    return h[:N]
